# baseline (speedup 1.0000x reference)
import functools

import jax
import jax.numpy as jnp
from jax import lax
from jax.experimental import pallas as pl
from jax.experimental.pallas import tpu as pltpu
from jax.experimental.pallas import tpu_sc as plsc

NC = 2
NS = 16
NW = NC * NS
L = 16
CE = 64


def _rsqrt_sc(x):
    i = plsc.bitcast(x, jnp.int32)
    i = jnp.int32(0x5F3759DF) - lax.shift_right_arithmetic(i, 1)
    y = plsc.bitcast(i, jnp.float32)
    for _ in range(3):
        y = y * (1.5 - 0.5 * x * y * y)
    return y


@functools.lru_cache(maxsize=None)
def _make_sca(n, np_, e, dp):
    nct = e // CE
    bc = nct // NW
    extra = nct - bc * NW
    mc = bc + (1 if extra else 0)
    mesh = plsc.VectorSubcoreMesh(core_axis_name="c", subcore_axis_name="s")

    @functools.partial(
        pl.kernel,
        mesh=mesh,
        compiler_params=pltpu.CompilerParams(use_tc_tiling_on_sc=False, needs_layout_passes=False),
        out_type=(
            jax.ShapeDtypeStruct((nct, CE), jnp.float32),
            jax.ShapeDtypeStruct((NW, np_), jnp.float32),
        ),
        scratch_types=[
            pltpu.VMEM((n,), jnp.float32),
            pltpu.VMEM((np_,), jnp.float32),
            pltpu.VMEM((mc, CE), jnp.int32),
            pltpu.VMEM((mc, CE), jnp.int32),
            pltpu.VMEM((mc, CE), jnp.float32),
            pltpu.VMEM((CE, dp), jnp.int32),
            pltpu.VMEM((CE, dp), jnp.int32),
            pltpu.VMEM((CE, dp), jnp.int32),
            pltpu.VMEM((CE, dp), jnp.int32),
            pltpu.SemaphoreType.DMA,
            pltpu.SemaphoreType.DMA,
            pltpu.SemaphoreType.DMA,
            pltpu.SemaphoreType.DMA,
        ],
    )
    def sca(xp_hbm, src2_hbm, dst2_hbm, kk_hbm, w2_hbm, degp_hbm,
            kk_v, ldeg, srcall, dstall, w_all,
            rs0, rs1, rd0, rd1, sems0, sems1, semd0, semd1):
        c = lax.axis_index("c")
        s = lax.axis_index("s")
        wid = c * NS + s
        nch = bc + jnp.where(wid < extra, 1, 0)
        cstart = wid * bc + jnp.minimum(wid, extra)
        rs = (rs0, rs1)
        rd = (rd0, rd1)
        sems = (sems0, sems1)
        semd = (semd0, semd1)

        pltpu.sync_copy(kk_hbm, kk_v)
        if extra:
            @pl.when(wid < extra)
            def _ld_hi():
                pltpu.sync_copy(src2_hbm.at[pl.ds(cstart, mc)], srcall)
                pltpu.sync_copy(dst2_hbm.at[pl.ds(cstart, mc)], dstall)

            @pl.when(wid >= extra)
            def _ld_lo():
                pltpu.sync_copy(src2_hbm.at[pl.ds(cstart, bc)],
                                srcall.at[pl.ds(0, bc)])
                pltpu.sync_copy(dst2_hbm.at[pl.ds(cstart, bc)],
                                dstall.at[pl.ds(0, bc)])
        else:
            pltpu.sync_copy(src2_hbm.at[pl.ds(cstart, bc)], srcall)
            pltpu.sync_copy(dst2_hbm.at[pl.ds(cstart, bc)], dstall)

        def zbody(i, carry):
            ldeg[pl.ds(i * L, L)] = jnp.zeros((L,), jnp.float32)
            return carry
        lax.fori_loop(0, np_ // L, zbody, 0)

        def start_gather(ci, b):
            pltpu.async_copy(xp_hbm.at[srcall.at[ci]], rs[b], sems[b])
            pltpu.async_copy(xp_hbm.at[dstall.at[ci]], rd[b], semd[b])

        @pl.when(nch > 0)
        def _pro():
            start_gather(0, 0)

        lanes16 = lax.iota(jnp.int32, L)

        def process(b, ci):
            rsb = rs[b]
            rdb = rd[b]
            pltpu.make_async_copy(xp_hbm.at[srcall.at[ci]], rsb,
                                  sems[b]).wait()
            pltpu.make_async_copy(xp_hbm.at[dstall.at[ci]], rdb,
                                  semd[b]).wait()

            @pl.when(ci + 1 < nch)
            def _pre():
                start_gather(ci + 1, 1 - b)

            for off in range(0, CE, L):
                lanes = lanes16 + off

                def dbody(cb, accs):
                    a0, a1 = accs
                    cbase = cb * L
                    for k in range(L):
                        col = cbase + ((lanes16 + k) & 15)
                        ps = plsc.load_gather(rsb, [lanes, col])
                        pd = plsc.load_gather(rdb, [lanes, col])
                        t = (plsc.bitcast(ps, jnp.bfloat16)
                             - plsc.bitcast(pd, jnp.bfloat16))
                        t1, t2 = plsc.unpack(
                            t, format=plsc.PackFormat.INTERLEAVED,
                            preferred_element_type=jnp.float32)
                        a0 = a0 + t1 * t1
                        a1 = a1 + t2 * t2
                    return (a0, a1)
                zz = jnp.zeros((L,), jnp.float32)
                a0, a1 = lax.fori_loop(0, dp // L, dbody, (zz, zz))
                disp = (a0 + a1) * (1.0 / (2 * dp))
                sv = srcall[ci, pl.ds(off, L)]
                dv = dstall[ci, pl.ds(off, L)]
                ks = plsc.load_gather(kk_v, [sv])
                kd = plsc.load_gather(kk_v, [dv])
                ww = jnp.exp(-disp) * _rsqrt_sc(ks * kd + 1.0)
                w_all[ci, pl.ds(off, L)] = ww
                occ = jnp.zeros((L,), jnp.int32)
                for shift in range(1, L):
                    prev = plsc.load_gather(
                        dstall,
                        [jnp.full((L,), ci, jnp.int32),
                         off + jnp.maximum(lanes16 - shift, 0)])
                    occ = occ + jnp.where(
                        (prev == dv) & (lanes16 >= shift), 1, 0)
                for k in range(L):
                    plsc.addupdate_scatter(ldeg, [dv], ww, mask=occ == k)

        def pair(p, carry):
            process(0, 2 * p)

            @pl.when(2 * p + 1 < nch)
            def _odd():
                process(1, 2 * p + 1)
            return carry
        lax.fori_loop(0, (nch + 1) // 2, pair, 0)

        if extra:
            @pl.when(wid < extra)
            def _st_hi():
                pltpu.sync_copy(w_all, w2_hbm.at[pl.ds(cstart, mc)])

            @pl.when(wid >= extra)
            def _st_lo():
                pltpu.sync_copy(w_all.at[pl.ds(0, bc)],
                                w2_hbm.at[pl.ds(cstart, bc)])
        else:
            pltpu.sync_copy(w_all, w2_hbm.at[pl.ds(cstart, bc)])
        pltpu.sync_copy(ldeg, degp_hbm.at[wid])

    return sca


@functools.lru_cache(maxsize=None)
def _make_scb(n, np_, e, f):
    per_core_ch = e // NC // CE
    bc = per_core_ch // NS
    extra = per_core_ch - bc * NS
    mc = bc + (1 if extra else 0)
    nz = n // NS
    mesh = plsc.VectorSubcoreMesh(core_axis_name="c", subcore_axis_name="s")

    @functools.partial(
        pl.kernel,
        mesh=mesh,
        compiler_params=pltpu.CompilerParams(use_tc_tiling_on_sc=False, needs_layout_passes=False),
        out_type=jax.ShapeDtypeStruct((f * NC * n, 128), jnp.float32),
        scratch_types=[
            pltpu.VMEM((n,), jnp.float32),
            pltpu.VMEM((mc, CE), jnp.int32),
            pltpu.VMEM((mc, CE), jnp.int32),
            pltpu.VMEM((mc, CE), jnp.float32),
            pltpu.VMEM((CE, 64), jnp.int32),
            pltpu.VMEM((CE, 64), jnp.int32),
            pltpu.VMEM((CE, 128), jnp.float32),
            pltpu.VMEM((CE, 128), jnp.float32),
            pltpu.VMEM_SHARED((n, 128), jnp.float32),
            pltpu.SemaphoreType.DMA,
            pltpu.SemaphoreType.DMA,
            pltpu.SemaphoreType.DMA,
        ],
    )
    def scb(hc_hbm, src2_hbm, dst2_hbm, w2_hbm, dinv_hbm, zeros_hbm, out_hbm,
            dinv_v, srcall, dstall, w_all, gb0, gb1, rows0, rows1, acc_sp,
            semg0, semg1, semsc):
        c = lax.axis_index("c")
        s = lax.axis_index("s")
        nch = bc + jnp.where(s < extra, 1, 0)
        cstart = c * per_core_ch + s * bc + jnp.minimum(s, extra)
        pltpu.sync_copy(dinv_hbm.at[pl.ds(0, n)], dinv_v)
        if extra:
            @pl.when(s < extra)
            def _ld_hi():
                pltpu.sync_copy(src2_hbm.at[pl.ds(cstart, bc + 1)], srcall)
                pltpu.sync_copy(dst2_hbm.at[pl.ds(cstart, bc + 1)], dstall)
                pltpu.sync_copy(w2_hbm.at[pl.ds(cstart, bc + 1)], w_all)

            @pl.when(s >= extra)
            def _ld_lo():
                pltpu.sync_copy(src2_hbm.at[pl.ds(cstart, bc)],
                                srcall.at[pl.ds(0, bc)])
                pltpu.sync_copy(dst2_hbm.at[pl.ds(cstart, bc)],
                                dstall.at[pl.ds(0, bc)])
                pltpu.sync_copy(w2_hbm.at[pl.ds(cstart, bc)],
                                w_all.at[pl.ds(0, bc)])
        else:
            pltpu.sync_copy(src2_hbm.at[pl.ds(cstart, bc)], srcall)
            pltpu.sync_copy(dst2_hbm.at[pl.ds(cstart, bc)], dstall)
            pltpu.sync_copy(w2_hbm.at[pl.ds(cstart, bc)], w_all)

        def ewchunk(ci, carry):
            for off in range(0, CE, L):
                sv = srcall[ci, pl.ds(off, L)]
                dv = dstall[ci, pl.ds(off, L)]
                ew = (plsc.load_gather(dinv_v, [sv])
                      * w_all[ci, pl.ds(off, L)]
                      * plsc.load_gather(dinv_v, [dv]))
                w_all[ci, pl.ds(off, L)] = ew
            return carry
        lax.fori_loop(0, nch, ewchunk, 0)

        gb = (gb0, gb1)
        rows = (rows0, rows1)
        semg = (semg0, semg1)

        for fc in range(f):
            if fc > 0:
                def sfchunk(ci, carry):
                    for off in range(0, CE, L):
                        srcall[ci, pl.ds(off, L)] = (
                            srcall[ci, pl.ds(off, L)] + n)
                    return carry
                lax.fori_loop(0, nch, sfchunk, 0)

            @pl.when(nch > 0)
            def _pro():
                pltpu.async_copy(hc_hbm.at[srcall.at[0]], gb[0], semg[0])
            pltpu.sync_copy(zeros_hbm.at[pl.ds(s * nz, nz)],
                            acc_sp.at[pl.ds(s * nz, nz)])
            plsc.subcore_barrier()

            def process(b, ci):
                gbb = gb[b]
                rb = rows[b]
                pltpu.make_async_copy(
                    hc_hbm.at[srcall.at[ci]], gbb, semg[b]).wait()

                @pl.when(ci + 1 < nch)
                def _pre():
                    pltpu.async_copy(hc_hbm.at[srcall.at[ci + 1]],
                                     gb[1 - b], semg[1 - b])

                @pl.when(ci >= 2)
                def _drain():
                    pltpu.make_async_copy(
                        rb, acc_sp.at[dstall.at[ci - 2]], semsc).wait()

                def rbody(rbi, carry2):
                    ewv = w_all[ci, pl.ds(rbi * L, L)]
                    for lane in range(L):
                        sc = ewv[lane]
                        r = rbi * L + lane
                        for j in range(64 // L):
                            p = gbb[r, pl.ds(j * L, L)]
                            t1, t2 = plsc.unpack(
                                plsc.bitcast(p, jnp.bfloat16),
                                format=plsc.PackFormat.INTERLEAVED,
                                preferred_element_type=jnp.float32)
                            rb[r, pl.ds(j * L, L)] = t1 * sc
                            rb[r, pl.ds(64 + j * L, L)] = t2 * sc
                    return carry2
                lax.fori_loop(0, CE // L, rbody, 0)
                pltpu.async_copy(rb, acc_sp.at[dstall.at[ci]], semsc, add=True)

            def pair(p, carry):
                process(0, 2 * p)

                @pl.when(2 * p + 1 < nch)
                def _odd():
                    process(1, 2 * p + 1)
                return carry
            lax.fori_loop(0, (nch + 1) // 2, pair, 0)

            @pl.when(nch >= 2)
            def _drain_a():
                pltpu.make_async_copy(
                    rows[0], acc_sp.at[dstall.at[0]], semsc).wait()

            @pl.when(nch >= 1)
            def _drain_b():
                pltpu.make_async_copy(
                    rows[0], acc_sp.at[dstall.at[0]], semsc).wait()
            plsc.subcore_barrier()

            base_out = (fc * NC + c) * n
            pltpu.sync_copy(acc_sp.at[pl.ds(s * nz, nz)],
                            out_hbm.at[pl.ds(base_out + s * nz, nz)])
            plsc.subcore_barrier()

    return scb


@functools.lru_cache(maxsize=None)
def _make_matmul(n, k, f):
    rb = 2000
    nrb = n // rb

    def mmk(x_ref, w_ref, o_ref):
        o_ref[...] = jnp.dot(
            x_ref[...], w_ref[...].astype(jnp.bfloat16),
            preferred_element_type=jnp.float32).astype(jnp.bfloat16)

    return pl.pallas_call(
        mmk,
        grid=(f, nrb),
        in_specs=[
            pl.BlockSpec((rb, k), lambda fc, r: (r, 0)),
            pl.BlockSpec((k, 128), lambda fc, r: (0, fc)),
        ],
        out_specs=pl.BlockSpec(
            (rb, 128), lambda fc, r, _nrb=nrb: (fc * _nrb + r, 0)),
        out_shape=jax.ShapeDtypeStruct((f * n, 128), jnp.bfloat16),
    )


@functools.lru_cache(maxsize=None)
def _make_dinv(np_):
    def k(d_ref, o_ref):
        o_ref[...] = lax.rsqrt(
            jnp.sum(d_ref[...], axis=0, keepdims=True) + 1e-6)

    return pl.pallas_call(
        k, out_shape=jax.ShapeDtypeStruct((1, np_), jnp.float32))


@functools.lru_cache(maxsize=None)
def _make_bn_relu(n, f):
    def k(o_ref, b_ref, g_ref, be_ref, h_ref):
        m = o_ref[...]
        hs = m[0:n] + m[n:2 * n] + b_ref[0]
        mu = jnp.mean(hs, axis=0, keepdims=True)
        xc = hs - mu
        var = jnp.mean(xc * xc, axis=0, keepdims=True)
        y = xc * lax.rsqrt(var + 1e-5) * g_ref[0] + be_ref[0]
        h_ref[...] = jnp.maximum(y, 0.0)

    return pl.pallas_call(
        k,
        grid=(f,),
        in_specs=[
            pl.BlockSpec((2 * n, 128), lambda fc: (fc, 0)),
            pl.BlockSpec((1, 1, 128), lambda fc: (fc, 0, 0)),
            pl.BlockSpec((1, 1, 128), lambda fc: (fc, 0, 0)),
            pl.BlockSpec((1, 1, 128), lambda fc: (fc, 0, 0)),
        ],
        out_specs=pl.BlockSpec((n, 128), lambda fc: (0, fc)),
        out_shape=jax.ShapeDtypeStruct((n, f * 128), jnp.float32),
    )


@functools.lru_cache(maxsize=None)
def _make_final(n):
    def k(o_ref, b_ref, y_ref):
        m = o_ref[...]
        z = m[0:n] + m[n:2 * n] + b_ref[...]
        zm = z - jnp.max(z, axis=1, keepdims=True)
        y_ref[...] = zm - jnp.log(
            jnp.sum(jnp.exp(zm), axis=1, keepdims=True))

    return pl.pallas_call(
        k, out_shape=jax.ShapeDtypeStruct((n, 128), jnp.float32))


def kernel(x, adj_t, k_hop_nbrs, W1, b1, g1, be1, W2, b2, g2, be2, W3, b3):
    n, din = x.shape
    e = adj_t.shape[1]
    np_ = ((n + 1023) // 1024) * 1024
    src = adj_t[0]
    dst = adj_t[1]
    kk = k_hop_nbrs.astype(jnp.float32)
    zeros_nf = jnp.zeros((n, 128), jnp.float32)

    src2 = src.reshape(e // CE, CE)
    dst2 = dst.reshape(e // CE, CE)

    def layer(hin, W):
        d = hin.shape[1]
        f = W.shape[1] // 128
        hb = hin.astype(jnp.bfloat16)
        xp = lax.bitcast_convert_type(hb.reshape(n, d // 2, 2), jnp.int32)
        w2, degp = _make_sca(n, np_, e, d // 2)(xp, src2, dst2, kk)
        hcb = _make_matmul(n, d, f)(hb, W)
        hcp = lax.bitcast_convert_type(
            jnp.swapaxes(hcb.reshape(f * n, 2, 64), 1, 2), jnp.int32)
        dinv = _make_dinv(np_)(degp).reshape(np_)
        return _make_scb(n, np_, e, f)(hcp, src2, dst2, w2, dinv, zeros_nf)

    o1 = layer(x, W1)
    h1 = _make_bn_relu(n, W1.shape[1] // 128)(
        o1, b1.reshape(-1, 1, 128), g1.reshape(-1, 1, 128),
        be1.reshape(-1, 1, 128))
    o2 = layer(h1, W2)
    h2 = _make_bn_relu(n, W2.shape[1] // 128)(
        o2, b2.reshape(-1, 1, 128), g2.reshape(-1, 1, 128),
        be2.reshape(-1, 1, 128))
    o3 = layer(h2, W3)
    return _make_final(n)(o3, b3.reshape(1, 128))


# speedup vs baseline: 1.3423x; 1.3423x over previous
import functools

import jax
import jax.numpy as jnp
from jax import lax
from jax.experimental import pallas as pl
from jax.experimental.pallas import tpu as pltpu
from jax.experimental.pallas import tpu_sc as plsc

NC = 2
NS = 16
NW = NC * NS
L = 16
CE = 64


def _rsqrt_sc(x):
    i = plsc.bitcast(x, jnp.int32)
    i = jnp.int32(0x5F3759DF) - lax.shift_right_arithmetic(i, 1)
    y = plsc.bitcast(i, jnp.float32)
    for _ in range(3):
        y = y * (1.5 - 0.5 * x * y * y)
    return y


@functools.lru_cache(maxsize=None)
def _make_sca(n, np_, e, dp):
    nct = e // CE
    bc = nct // NW
    extra = nct - bc * NW
    mc = bc + (1 if extra else 0)
    mesh = plsc.VectorSubcoreMesh(core_axis_name="c", subcore_axis_name="s")

    @functools.partial(
        pl.kernel,
        mesh=mesh,
        compiler_params=pltpu.CompilerParams(use_tc_tiling_on_sc=False, needs_layout_passes=False),
        out_type=(
            jax.ShapeDtypeStruct((nct, CE), jnp.float32),
            jax.ShapeDtypeStruct((NW, np_), jnp.float32),
        ),
        scratch_types=[
            pltpu.VMEM((n,), jnp.float32),
            pltpu.VMEM((np_,), jnp.float32),
            pltpu.VMEM((mc, CE), jnp.int32),
            pltpu.VMEM((mc, CE), jnp.int32),
            pltpu.VMEM((mc, CE), jnp.float32),
            pltpu.VMEM((CE, dp), jnp.int32),
            pltpu.VMEM((CE, dp), jnp.int32),
            pltpu.VMEM((CE, dp), jnp.int32),
            pltpu.VMEM((CE, dp), jnp.int32),
            pltpu.SemaphoreType.DMA,
            pltpu.SemaphoreType.DMA,
            pltpu.SemaphoreType.DMA,
            pltpu.SemaphoreType.DMA,
        ],
    )
    def sca(xp_hbm, src2_hbm, dst2_hbm, kk_hbm, w2_hbm, degp_hbm,
            kk_v, ldeg, srcall, dstall, w_all,
            rs0, rs1, rd0, rd1, sems0, sems1, semd0, semd1):
        c = lax.axis_index("c")
        s = lax.axis_index("s")
        wid = c * NS + s
        nch = bc + jnp.where(wid < extra, 1, 0)
        cstart = wid * bc + jnp.minimum(wid, extra)
        rs = (rs0, rs1)
        rd = (rd0, rd1)
        sems = (sems0, sems1)
        semd = (semd0, semd1)

        pltpu.sync_copy(kk_hbm, kk_v)
        if extra:
            @pl.when(wid < extra)
            def _ld_hi():
                pltpu.sync_copy(src2_hbm.at[pl.ds(cstart, mc)], srcall)
                pltpu.sync_copy(dst2_hbm.at[pl.ds(cstart, mc)], dstall)

            @pl.when(wid >= extra)
            def _ld_lo():
                pltpu.sync_copy(src2_hbm.at[pl.ds(cstart, bc)],
                                srcall.at[pl.ds(0, bc)])
                pltpu.sync_copy(dst2_hbm.at[pl.ds(cstart, bc)],
                                dstall.at[pl.ds(0, bc)])
        else:
            pltpu.sync_copy(src2_hbm.at[pl.ds(cstart, bc)], srcall)
            pltpu.sync_copy(dst2_hbm.at[pl.ds(cstart, bc)], dstall)

        def zbody(i, carry):
            ldeg[pl.ds(i * L, L)] = jnp.zeros((L,), jnp.float32)
            return carry
        lax.fori_loop(0, np_ // L, zbody, 0)

        def start_gather(ci, b):
            pltpu.async_copy(xp_hbm.at[srcall.at[ci]], rs[b], sems[b])
            pltpu.async_copy(xp_hbm.at[dstall.at[ci]], rd[b], semd[b])

        @pl.when(nch > 0)
        def _pro():
            start_gather(0, 0)

        lanes16 = lax.iota(jnp.int32, L)

        def process(b, ci):
            rsb = rs[b]
            rdb = rd[b]
            pltpu.make_async_copy(xp_hbm.at[srcall.at[ci]], rsb,
                                  sems[b]).wait()
            pltpu.make_async_copy(xp_hbm.at[dstall.at[ci]], rdb,
                                  semd[b]).wait()

            @pl.when(ci + 1 < nch)
            def _pre():
                start_gather(ci + 1, 1 - b)

            for off in range(0, CE, L):
                lanes = lanes16 + off

                def dbody(cb, accs):
                    a0, a1 = accs
                    cbase = cb * L
                    for k in range(L):
                        col = cbase + ((lanes16 + k) & 15)
                        ps = plsc.load_gather(rsb, [lanes, col])
                        pd = plsc.load_gather(rdb, [lanes, col])
                        t = (plsc.bitcast(ps, jnp.bfloat16)
                             - plsc.bitcast(pd, jnp.bfloat16))
                        t1, t2 = plsc.unpack(
                            t, format=plsc.PackFormat.INTERLEAVED,
                            preferred_element_type=jnp.float32)
                        a0 = a0 + t1 * t1
                        a1 = a1 + t2 * t2
                    return (a0, a1)
                zz = jnp.zeros((L,), jnp.float32)
                a0, a1 = lax.fori_loop(0, dp // L, dbody, (zz, zz))
                disp = (a0 + a1) * (1.0 / (2 * dp))
                sv = srcall[ci, pl.ds(off, L)]
                dv = dstall[ci, pl.ds(off, L)]
                ks = plsc.load_gather(kk_v, [sv])
                kd = plsc.load_gather(kk_v, [dv])
                ww = jnp.exp(-disp) * _rsqrt_sc(ks * kd + 1.0)
                w_all[ci, pl.ds(off, L)] = ww
                occ = jnp.zeros((L,), jnp.int32)
                for shift in range(1, L):
                    prev = plsc.load_gather(
                        dstall,
                        [jnp.full((L,), ci, jnp.int32),
                         off + jnp.maximum(lanes16 - shift, 0)])
                    occ = occ + jnp.where(
                        (prev == dv) & (lanes16 >= shift), 1, 0)
                for k in range(L):
                    plsc.addupdate_scatter(ldeg, [dv], ww, mask=occ == k)

        def pair(p, carry):
            process(0, 2 * p)

            @pl.when(2 * p + 1 < nch)
            def _odd():
                process(1, 2 * p + 1)
            return carry
        lax.fori_loop(0, (nch + 1) // 2, pair, 0)

        if extra:
            @pl.when(wid < extra)
            def _st_hi():
                pltpu.sync_copy(w_all, w2_hbm.at[pl.ds(cstart, mc)])

            @pl.when(wid >= extra)
            def _st_lo():
                pltpu.sync_copy(w_all.at[pl.ds(0, bc)],
                                w2_hbm.at[pl.ds(cstart, bc)])
        else:
            pltpu.sync_copy(w_all, w2_hbm.at[pl.ds(cstart, bc)])
        pltpu.sync_copy(ldeg, degp_hbm.at[wid])

    return sca


@functools.lru_cache(maxsize=None)
def _make_scb(n, np_, e, f):
    per_core_ch = e // NC // CE
    bc = per_core_ch // NS
    extra = per_core_ch - bc * NS
    mc = bc + (1 if extra else 0)
    nz = n // NS
    mesh = plsc.VectorSubcoreMesh(core_axis_name="c", subcore_axis_name="s")

    @functools.partial(
        pl.kernel,
        mesh=mesh,
        compiler_params=pltpu.CompilerParams(use_tc_tiling_on_sc=False, needs_layout_passes=False),
        out_type=jax.ShapeDtypeStruct((f * NC * n, 128), jnp.float32),
        scratch_types=[
            pltpu.VMEM((n,), jnp.float32),
            pltpu.VMEM((mc, CE), jnp.int32),
            pltpu.VMEM((mc, CE), jnp.int32),
            pltpu.VMEM((mc, CE), jnp.float32),
            pltpu.VMEM((CE, 128), jnp.float32),
            pltpu.VMEM((CE, 128), jnp.float32),
            pltpu.VMEM((CE, 128), jnp.float32),
            pltpu.VMEM_SHARED((n, 128), jnp.float32),
            pltpu.SemaphoreType.DMA,
            pltpu.SemaphoreType.DMA,
            pltpu.SemaphoreType.DMA,
            pltpu.SemaphoreType.DMA,
        ],
    )
    def scb(hc_hbm, src2_hbm, dst2_hbm, w2_hbm, dinv_hbm, zeros_hbm, out_hbm,
            dinv_v, srcall, dstall, w_all, rows0, rows1, rows2, acc_sp,
            semg0, semg1, semg2, semsc):
        c = lax.axis_index("c")
        s = lax.axis_index("s")
        nch = bc + jnp.where(s < extra, 1, 0)
        cstart = c * per_core_ch + s * bc + jnp.minimum(s, extra)
        pltpu.sync_copy(dinv_hbm.at[pl.ds(0, n)], dinv_v)
        if extra:
            @pl.when(s < extra)
            def _ld_hi():
                pltpu.sync_copy(src2_hbm.at[pl.ds(cstart, bc + 1)], srcall)
                pltpu.sync_copy(dst2_hbm.at[pl.ds(cstart, bc + 1)], dstall)
                pltpu.sync_copy(w2_hbm.at[pl.ds(cstart, bc + 1)], w_all)

            @pl.when(s >= extra)
            def _ld_lo():
                pltpu.sync_copy(src2_hbm.at[pl.ds(cstart, bc)],
                                srcall.at[pl.ds(0, bc)])
                pltpu.sync_copy(dst2_hbm.at[pl.ds(cstart, bc)],
                                dstall.at[pl.ds(0, bc)])
                pltpu.sync_copy(w2_hbm.at[pl.ds(cstart, bc)],
                                w_all.at[pl.ds(0, bc)])
        else:
            pltpu.sync_copy(src2_hbm.at[pl.ds(cstart, bc)], srcall)
            pltpu.sync_copy(dst2_hbm.at[pl.ds(cstart, bc)], dstall)
            pltpu.sync_copy(w2_hbm.at[pl.ds(cstart, bc)], w_all)

        def ewchunk(ci, carry):
            for off in range(0, CE, L):
                sv = srcall[ci, pl.ds(off, L)]
                dv = dstall[ci, pl.ds(off, L)]
                ew = (plsc.load_gather(dinv_v, [sv])
                      * w_all[ci, pl.ds(off, L)]
                      * plsc.load_gather(dinv_v, [dv]))
                w_all[ci, pl.ds(off, L)] = ew
            return carry
        lax.fori_loop(0, nch, ewchunk, 0)

        rows = (rows0, rows1, rows2)
        semg = (semg0, semg1, semg2)

        for fc in range(f):
            if fc > 0:
                def sfchunk(ci, carry):
                    for off in range(0, CE, L):
                        srcall[ci, pl.ds(off, L)] = (
                            srcall[ci, pl.ds(off, L)] + n)
                    return carry
                lax.fori_loop(0, nch, sfchunk, 0)

            @pl.when(nch > 0)
            def _pro():
                pltpu.async_copy(hc_hbm.at[srcall.at[0]], rows[0], semg[0])

            @pl.when(nch > 1)
            def _pro2():
                pltpu.async_copy(hc_hbm.at[srcall.at[1]], rows[1], semg[1])
            pltpu.sync_copy(zeros_hbm.at[pl.ds(s * nz, nz)],
                            acc_sp.at[pl.ds(s * nz, nz)])
            plsc.subcore_barrier()

            def process(b, ci):
                rb = rows[b]
                nb = rows[(b + 2) % 3]
                pltpu.make_async_copy(
                    hc_hbm.at[srcall.at[ci]], rb, semg[b]).wait()

                def rbody(rbi, carry2):
                    ewv = w_all[ci, pl.ds(rbi * L, L)]
                    for lane in range(L):
                        sc = ewv[lane]
                        r = rbi * L + lane
                        for j in range(128 // L):
                            rb[r, pl.ds(j * L, L)] = (
                                rb[r, pl.ds(j * L, L)] * sc)
                    return carry2
                lax.fori_loop(0, CE // L, rbody, 0)

                @pl.when(ci >= 1)
                def _drain():
                    pltpu.make_async_copy(
                        nb, acc_sp.at[dstall.at[ci - 1]], semsc).wait()

                @pl.when(ci + 2 < nch)
                def _pre():
                    pltpu.async_copy(hc_hbm.at[srcall.at[ci + 2]],
                                     nb, semg[(b + 2) % 3])
                pltpu.async_copy(rb, acc_sp.at[dstall.at[ci]], semsc,
                                 add=True)

            def triple(p, carry):
                process(0, 3 * p)

                @pl.when(3 * p + 1 < nch)
                def _t1():
                    process(1, 3 * p + 1)

                @pl.when(3 * p + 2 < nch)
                def _t2():
                    process(2, 3 * p + 2)
                return carry
            lax.fori_loop(0, (nch + 2) // 3, triple, 0)

            @pl.when(nch >= 1)
            def _drain_b():
                pltpu.make_async_copy(
                    rows[0], acc_sp.at[dstall.at[0]], semsc).wait()
            plsc.subcore_barrier()

            base_out = (fc * NC + c) * n
            pltpu.sync_copy(acc_sp.at[pl.ds(s * nz, nz)],
                            out_hbm.at[pl.ds(base_out + s * nz, nz)])
            plsc.subcore_barrier()

    return scb


@functools.lru_cache(maxsize=None)
def _make_matmul(n, k, f):
    rb = 2000
    nrb = n // rb

    def mmk(x_ref, w_ref, o_ref):
        o_ref[...] = jnp.dot(
            x_ref[...], w_ref[...].astype(jnp.bfloat16),
            preferred_element_type=jnp.float32)

    return pl.pallas_call(
        mmk,
        grid=(f, nrb),
        in_specs=[
            pl.BlockSpec((rb, k), lambda fc, r: (r, 0)),
            pl.BlockSpec((k, 128), lambda fc, r: (0, fc)),
        ],
        out_specs=pl.BlockSpec(
            (rb, 128), lambda fc, r, _nrb=nrb: (fc * _nrb + r, 0)),
        out_shape=jax.ShapeDtypeStruct((f * n, 128), jnp.float32),
    )


@functools.lru_cache(maxsize=None)
def _make_dinv(np_):
    def k(d_ref, o_ref):
        o_ref[...] = lax.rsqrt(
            jnp.sum(d_ref[...], axis=0, keepdims=True) + 1e-6)

    return pl.pallas_call(
        k, out_shape=jax.ShapeDtypeStruct((1, np_), jnp.float32))


@functools.lru_cache(maxsize=None)
def _make_bn_relu(n, f):
    def k(o_ref, b_ref, g_ref, be_ref, h_ref):
        m = o_ref[...]
        hs = m[0:n] + m[n:2 * n] + b_ref[0]
        mu = jnp.mean(hs, axis=0, keepdims=True)
        xc = hs - mu
        var = jnp.mean(xc * xc, axis=0, keepdims=True)
        y = xc * lax.rsqrt(var + 1e-5) * g_ref[0] + be_ref[0]
        h_ref[...] = jnp.maximum(y, 0.0)

    return pl.pallas_call(
        k,
        grid=(f,),
        in_specs=[
            pl.BlockSpec((2 * n, 128), lambda fc: (fc, 0)),
            pl.BlockSpec((1, 1, 128), lambda fc: (fc, 0, 0)),
            pl.BlockSpec((1, 1, 128), lambda fc: (fc, 0, 0)),
            pl.BlockSpec((1, 1, 128), lambda fc: (fc, 0, 0)),
        ],
        out_specs=pl.BlockSpec((n, 128), lambda fc: (0, fc)),
        out_shape=jax.ShapeDtypeStruct((n, f * 128), jnp.float32),
    )


@functools.lru_cache(maxsize=None)
def _make_final(n):
    def k(o_ref, b_ref, y_ref):
        m = o_ref[...]
        z = m[0:n] + m[n:2 * n] + b_ref[...]
        zm = z - jnp.max(z, axis=1, keepdims=True)
        y_ref[...] = zm - jnp.log(
            jnp.sum(jnp.exp(zm), axis=1, keepdims=True))

    return pl.pallas_call(
        k, out_shape=jax.ShapeDtypeStruct((n, 128), jnp.float32))


def kernel(x, adj_t, k_hop_nbrs, W1, b1, g1, be1, W2, b2, g2, be2, W3, b3):
    n, din = x.shape
    e = adj_t.shape[1]
    np_ = ((n + 1023) // 1024) * 1024
    src = adj_t[0]
    dst = adj_t[1]
    kk = k_hop_nbrs.astype(jnp.float32)
    zeros_nf = jnp.zeros((n, 128), jnp.float32)

    src2 = src.reshape(e // CE, CE)
    dst2 = dst.reshape(e // CE, CE)

    def layer(hin, W):
        d = hin.shape[1]
        f = W.shape[1] // 128
        hb = hin.astype(jnp.bfloat16)
        xp = lax.bitcast_convert_type(hb.reshape(n, d // 2, 2), jnp.int32)
        w2, degp = _make_sca(n, np_, e, d // 2)(xp, src2, dst2, kk)
        hc = _make_matmul(n, d, f)(hb, W)
        dinv = _make_dinv(np_)(degp).reshape(np_)
        return _make_scb(n, np_, e, f)(hc, src2, dst2, w2, dinv, zeros_nf)

    o1 = layer(x, W1)
    h1 = _make_bn_relu(n, W1.shape[1] // 128)(
        o1, b1.reshape(-1, 1, 128), g1.reshape(-1, 1, 128),
        be1.reshape(-1, 1, 128))
    o2 = layer(h1, W2)
    h2 = _make_bn_relu(n, W2.shape[1] // 128)(
        o2, b2.reshape(-1, 1, 128), g2.reshape(-1, 1, 128),
        be2.reshape(-1, 1, 128))
    o3 = layer(h2, W3)
    return _make_final(n)(o3, b3.reshape(1, 128))


# speedup vs baseline: 1.3502x; 1.0059x over previous
import functools

import jax
import jax.numpy as jnp
from jax import lax
from jax.experimental import pallas as pl
from jax.experimental.pallas import tpu as pltpu
from jax.experimental.pallas import tpu_sc as plsc

NC = 2
NS = 16
NW = NC * NS
L = 16
CE = 64


def _rsqrt_sc(x):
    i = plsc.bitcast(x, jnp.int32)
    i = jnp.int32(0x5F3759DF) - lax.shift_right_arithmetic(i, 1)
    y = plsc.bitcast(i, jnp.float32)
    for _ in range(3):
        y = y * (1.5 - 0.5 * x * y * y)
    return y


@functools.lru_cache(maxsize=None)
def _make_sca(n, np_, e, dp):
    nct = e // CE
    bc = nct // NW
    extra = nct - bc * NW
    mc = bc + (1 if extra else 0)
    mesh = plsc.VectorSubcoreMesh(core_axis_name="c", subcore_axis_name="s")

    @functools.partial(
        pl.kernel,
        mesh=mesh,
        compiler_params=pltpu.CompilerParams(use_tc_tiling_on_sc=False, needs_layout_passes=False),
        out_type=(
            jax.ShapeDtypeStruct((nct, CE), jnp.float32),
            jax.ShapeDtypeStruct((NW, np_), jnp.float32),
        ),
        scratch_types=[
            pltpu.VMEM((n,), jnp.float32),
            pltpu.VMEM((np_,), jnp.float32),
            pltpu.VMEM((mc, CE), jnp.int32),
            pltpu.VMEM((mc, CE), jnp.int32),
            pltpu.VMEM((mc, CE), jnp.float32),
            pltpu.VMEM((CE, dp), jnp.int32),
            pltpu.VMEM((CE, dp), jnp.int32),
            pltpu.VMEM((CE, dp), jnp.int32),
            pltpu.VMEM((CE, dp), jnp.int32),
            pltpu.SemaphoreType.DMA,
            pltpu.SemaphoreType.DMA,
            pltpu.SemaphoreType.DMA,
            pltpu.SemaphoreType.DMA,
        ],
    )
    def sca(xp_hbm, src2_hbm, dst2_hbm, kk_hbm, w2_hbm, degp_hbm,
            kk_v, ldeg, srcall, dstall, w_all,
            rs0, rs1, rd0, rd1, sems0, sems1, semd0, semd1):
        c = lax.axis_index("c")
        s = lax.axis_index("s")
        wid = c * NS + s
        nch = bc + jnp.where(wid < extra, 1, 0)
        cstart = wid * bc + jnp.minimum(wid, extra)
        rs = (rs0, rs1)
        rd = (rd0, rd1)
        sems = (sems0, sems1)
        semd = (semd0, semd1)

        pltpu.sync_copy(kk_hbm, kk_v)
        if extra:
            @pl.when(wid < extra)
            def _ld_hi():
                pltpu.sync_copy(src2_hbm.at[pl.ds(cstart, mc)], srcall)
                pltpu.sync_copy(dst2_hbm.at[pl.ds(cstart, mc)], dstall)

            @pl.when(wid >= extra)
            def _ld_lo():
                pltpu.sync_copy(src2_hbm.at[pl.ds(cstart, bc)],
                                srcall.at[pl.ds(0, bc)])
                pltpu.sync_copy(dst2_hbm.at[pl.ds(cstart, bc)],
                                dstall.at[pl.ds(0, bc)])
        else:
            pltpu.sync_copy(src2_hbm.at[pl.ds(cstart, bc)], srcall)
            pltpu.sync_copy(dst2_hbm.at[pl.ds(cstart, bc)], dstall)

        def zbody(i, carry):
            ldeg[pl.ds(i * L, L)] = jnp.zeros((L,), jnp.float32)
            return carry
        lax.fori_loop(0, np_ // L, zbody, 0)

        def start_gather(ci, b):
            pltpu.async_copy(xp_hbm.at[srcall.at[ci]], rs[b], sems[b])
            pltpu.async_copy(xp_hbm.at[dstall.at[ci]], rd[b], semd[b])

        @pl.when(nch > 0)
        def _pro():
            start_gather(0, 0)

        lanes16 = lax.iota(jnp.int32, L)

        def process(b, ci):
            rsb = rs[b]
            rdb = rd[b]
            pltpu.make_async_copy(xp_hbm.at[srcall.at[ci]], rsb,
                                  sems[b]).wait()
            pltpu.make_async_copy(xp_hbm.at[dstall.at[ci]], rdb,
                                  semd[b]).wait()

            @pl.when(ci + 1 < nch)
            def _pre():
                start_gather(ci + 1, 1 - b)

            for off in range(0, CE, L):
                lanes = lanes16 + off

                def dbody(cb, accs):
                    a0, a1 = accs
                    cbase = cb * L
                    for k in range(L):
                        col = cbase + ((lanes16 + k) & 15)
                        ps = plsc.load_gather(rsb, [lanes, col])
                        pd = plsc.load_gather(rdb, [lanes, col])
                        t = (plsc.bitcast(ps, jnp.bfloat16)
                             - plsc.bitcast(pd, jnp.bfloat16))
                        t1, t2 = plsc.unpack(
                            t, format=plsc.PackFormat.INTERLEAVED,
                            preferred_element_type=jnp.float32)
                        a0 = a0 + t1 * t1
                        a1 = a1 + t2 * t2
                    return (a0, a1)
                zz = jnp.zeros((L,), jnp.float32)
                a0, a1 = lax.fori_loop(0, dp // L, dbody, (zz, zz))
                disp = (a0 + a1) * (1.0 / (2 * dp))
                sv = srcall[ci, pl.ds(off, L)]
                dv = dstall[ci, pl.ds(off, L)]
                ks = plsc.load_gather(kk_v, [sv])
                kd = plsc.load_gather(kk_v, [dv])
                ww = jnp.exp(-disp) * _rsqrt_sc(ks * kd + 1.0)
                w_all[ci, pl.ds(off, L)] = ww
                occ = jnp.zeros((L,), jnp.int32)
                civ = jnp.full((L,), ci, jnp.int32)
                for shift in range(1, L):
                    prev = plsc.load_gather(
                        dstall, [civ, off + ((lanes16 - shift) & 15)])
                    occ = occ + jnp.where(
                        (prev == dv) & (lanes16 >= shift), 1, 0)

                def addk(k, carry2):
                    plsc.addupdate_scatter(ldeg, [dv], ww, mask=occ == k)
                    return carry2
                lax.fori_loop(0, jnp.max(occ) + 1, addk, 0)

        def pair(p, carry):
            process(0, 2 * p)

            @pl.when(2 * p + 1 < nch)
            def _odd():
                process(1, 2 * p + 1)
            return carry
        lax.fori_loop(0, (nch + 1) // 2, pair, 0)

        if extra:
            @pl.when(wid < extra)
            def _st_hi():
                pltpu.sync_copy(w_all, w2_hbm.at[pl.ds(cstart, mc)])

            @pl.when(wid >= extra)
            def _st_lo():
                pltpu.sync_copy(w_all.at[pl.ds(0, bc)],
                                w2_hbm.at[pl.ds(cstart, bc)])
        else:
            pltpu.sync_copy(w_all, w2_hbm.at[pl.ds(cstart, bc)])
        pltpu.sync_copy(ldeg, degp_hbm.at[wid])

    return sca


@functools.lru_cache(maxsize=None)
def _make_scb(n, np_, e, f):
    per_core_ch = e // NC // CE
    bc = per_core_ch // NS
    extra = per_core_ch - bc * NS
    mc = bc + (1 if extra else 0)
    nz = n // NS
    mesh = plsc.VectorSubcoreMesh(core_axis_name="c", subcore_axis_name="s")

    @functools.partial(
        pl.kernel,
        mesh=mesh,
        compiler_params=pltpu.CompilerParams(use_tc_tiling_on_sc=False, needs_layout_passes=False),
        out_type=jax.ShapeDtypeStruct((f * NC * n, 128), jnp.float32),
        scratch_types=[
            pltpu.VMEM((n,), jnp.float32),
            pltpu.VMEM((mc, CE), jnp.int32),
            pltpu.VMEM((mc, CE), jnp.int32),
            pltpu.VMEM((mc, CE), jnp.float32),
            pltpu.VMEM((CE, 128), jnp.float32),
            pltpu.VMEM((CE, 128), jnp.float32),
            pltpu.VMEM((CE, 128), jnp.float32),
            pltpu.VMEM_SHARED((n, 128), jnp.float32),
            pltpu.SemaphoreType.DMA,
            pltpu.SemaphoreType.DMA,
            pltpu.SemaphoreType.DMA,
            pltpu.SemaphoreType.DMA,
        ],
    )
    def scb(hc_hbm, src2_hbm, dst2_hbm, w2_hbm, dinv_hbm, zeros_hbm, out_hbm,
            dinv_v, srcall, dstall, w_all, rows0, rows1, rows2, acc_sp,
            semg0, semg1, semg2, semsc):
        c = lax.axis_index("c")
        s = lax.axis_index("s")
        nch = bc + jnp.where(s < extra, 1, 0)
        cstart = c * per_core_ch + s * bc + jnp.minimum(s, extra)
        pltpu.sync_copy(dinv_hbm.at[pl.ds(0, n)], dinv_v)
        if extra:
            @pl.when(s < extra)
            def _ld_hi():
                pltpu.sync_copy(src2_hbm.at[pl.ds(cstart, bc + 1)], srcall)
                pltpu.sync_copy(dst2_hbm.at[pl.ds(cstart, bc + 1)], dstall)
                pltpu.sync_copy(w2_hbm.at[pl.ds(cstart, bc + 1)], w_all)

            @pl.when(s >= extra)
            def _ld_lo():
                pltpu.sync_copy(src2_hbm.at[pl.ds(cstart, bc)],
                                srcall.at[pl.ds(0, bc)])
                pltpu.sync_copy(dst2_hbm.at[pl.ds(cstart, bc)],
                                dstall.at[pl.ds(0, bc)])
                pltpu.sync_copy(w2_hbm.at[pl.ds(cstart, bc)],
                                w_all.at[pl.ds(0, bc)])
        else:
            pltpu.sync_copy(src2_hbm.at[pl.ds(cstart, bc)], srcall)
            pltpu.sync_copy(dst2_hbm.at[pl.ds(cstart, bc)], dstall)
            pltpu.sync_copy(w2_hbm.at[pl.ds(cstart, bc)], w_all)

        def ewchunk(ci, carry):
            for off in range(0, CE, L):
                sv = srcall[ci, pl.ds(off, L)]
                dv = dstall[ci, pl.ds(off, L)]
                ew = (plsc.load_gather(dinv_v, [sv])
                      * w_all[ci, pl.ds(off, L)]
                      * plsc.load_gather(dinv_v, [dv]))
                w_all[ci, pl.ds(off, L)] = ew
            return carry
        lax.fori_loop(0, nch, ewchunk, 0)

        rows = (rows0, rows1, rows2)
        semg = (semg0, semg1, semg2)

        for fc in range(f):
            if fc > 0:
                def sfchunk(ci, carry):
                    for off in range(0, CE, L):
                        srcall[ci, pl.ds(off, L)] = (
                            srcall[ci, pl.ds(off, L)] + n)
                    return carry
                lax.fori_loop(0, nch, sfchunk, 0)

            @pl.when(nch > 0)
            def _pro():
                pltpu.async_copy(hc_hbm.at[srcall.at[0]], rows[0], semg[0])

            @pl.when(nch > 1)
            def _pro2():
                pltpu.async_copy(hc_hbm.at[srcall.at[1]], rows[1], semg[1])
            pltpu.sync_copy(zeros_hbm.at[pl.ds(s * nz, nz)],
                            acc_sp.at[pl.ds(s * nz, nz)])
            plsc.subcore_barrier()

            def process(b, ci):
                rb = rows[b]
                nb = rows[(b + 2) % 3]
                pltpu.make_async_copy(
                    hc_hbm.at[srcall.at[ci]], rb, semg[b]).wait()

                def rbody(rbi, carry2):
                    ewv = w_all[ci, pl.ds(rbi * L, L)]
                    for lane in range(L):
                        sc = ewv[lane]
                        r = rbi * L + lane
                        for j in range(128 // L):
                            rb[r, pl.ds(j * L, L)] = (
                                rb[r, pl.ds(j * L, L)] * sc)
                    return carry2
                lax.fori_loop(0, CE // L, rbody, 0)

                @pl.when(ci >= 1)
                def _drain():
                    pltpu.make_async_copy(
                        nb, acc_sp.at[dstall.at[ci - 1]], semsc).wait()

                @pl.when(ci + 2 < nch)
                def _pre():
                    pltpu.async_copy(hc_hbm.at[srcall.at[ci + 2]],
                                     nb, semg[(b + 2) % 3])
                pltpu.async_copy(rb, acc_sp.at[dstall.at[ci]], semsc,
                                 add=True)

            def triple(p, carry):
                process(0, 3 * p)

                @pl.when(3 * p + 1 < nch)
                def _t1():
                    process(1, 3 * p + 1)

                @pl.when(3 * p + 2 < nch)
                def _t2():
                    process(2, 3 * p + 2)
                return carry
            lax.fori_loop(0, (nch + 2) // 3, triple, 0)

            @pl.when(nch >= 1)
            def _drain_b():
                pltpu.make_async_copy(
                    rows[0], acc_sp.at[dstall.at[0]], semsc).wait()
            plsc.subcore_barrier()

            base_out = (fc * NC + c) * n
            pltpu.sync_copy(acc_sp.at[pl.ds(s * nz, nz)],
                            out_hbm.at[pl.ds(base_out + s * nz, nz)])
            plsc.subcore_barrier()

    return scb


@functools.lru_cache(maxsize=None)
def _make_matmul(n, k, f):
    rb = 2000
    nrb = n // rb

    def mmk(x_ref, w_ref, o_ref):
        o_ref[...] = jnp.dot(
            x_ref[...], w_ref[...].astype(jnp.bfloat16),
            preferred_element_type=jnp.float32)

    return pl.pallas_call(
        mmk,
        grid=(nrb, f),
        in_specs=[
            pl.BlockSpec((rb, k), lambda r, fc: (r, 0)),
            pl.BlockSpec((k, 128), lambda r, fc: (0, fc)),
        ],
        out_specs=pl.BlockSpec(
            (rb, 128), lambda r, fc, _nrb=nrb: (fc * _nrb + r, 0)),
        out_shape=jax.ShapeDtypeStruct((f * n, 128), jnp.float32),
    )


@functools.lru_cache(maxsize=None)
def _make_dinv(np_):
    def k(d_ref, o_ref):
        o_ref[...] = lax.rsqrt(
            jnp.sum(d_ref[...], axis=0, keepdims=True) + 1e-6)

    return pl.pallas_call(
        k, out_shape=jax.ShapeDtypeStruct((1, np_), jnp.float32))


@functools.lru_cache(maxsize=None)
def _make_bn_relu(n, f):
    def k(o_ref, b_ref, g_ref, be_ref, h_ref):
        m = o_ref[...]
        hs = m[0:n] + m[n:2 * n] + b_ref[0]
        mu = jnp.mean(hs, axis=0, keepdims=True)
        xc = hs - mu
        var = jnp.mean(xc * xc, axis=0, keepdims=True)
        y = xc * lax.rsqrt(var + 1e-5) * g_ref[0] + be_ref[0]
        h_ref[...] = jnp.maximum(y, 0.0)

    return pl.pallas_call(
        k,
        grid=(f,),
        in_specs=[
            pl.BlockSpec((2 * n, 128), lambda fc: (fc, 0)),
            pl.BlockSpec((1, 1, 128), lambda fc: (fc, 0, 0)),
            pl.BlockSpec((1, 1, 128), lambda fc: (fc, 0, 0)),
            pl.BlockSpec((1, 1, 128), lambda fc: (fc, 0, 0)),
        ],
        out_specs=pl.BlockSpec((n, 128), lambda fc: (0, fc)),
        out_shape=jax.ShapeDtypeStruct((n, f * 128), jnp.float32),
    )


@functools.lru_cache(maxsize=None)
def _make_final(n):
    def k(o_ref, b_ref, y_ref):
        m = o_ref[...]
        z = m[0:n] + m[n:2 * n] + b_ref[...]
        zm = z - jnp.max(z, axis=1, keepdims=True)
        y_ref[...] = zm - jnp.log(
            jnp.sum(jnp.exp(zm), axis=1, keepdims=True))

    return pl.pallas_call(
        k, out_shape=jax.ShapeDtypeStruct((n, 128), jnp.float32))


def kernel(x, adj_t, k_hop_nbrs, W1, b1, g1, be1, W2, b2, g2, be2, W3, b3):
    n, din = x.shape
    e = adj_t.shape[1]
    np_ = ((n + 1023) // 1024) * 1024
    src = adj_t[0]
    dst = adj_t[1]
    kk = k_hop_nbrs.astype(jnp.float32)
    zeros_nf = jnp.zeros((n, 128), jnp.float32)

    src2 = src.reshape(e // CE, CE)
    dst2 = dst.reshape(e // CE, CE)

    def layer(hin, W):
        d = hin.shape[1]
        f = W.shape[1] // 128
        hb = hin.astype(jnp.bfloat16)
        xp = lax.bitcast_convert_type(hb.reshape(n, d // 2, 2), jnp.int32)
        w2, degp = _make_sca(n, np_, e, d // 2)(xp, src2, dst2, kk)
        hc = _make_matmul(n, d, f)(hb, W)
        dinv = _make_dinv(np_)(degp).reshape(np_)
        return _make_scb(n, np_, e, f)(hc, src2, dst2, w2, dinv, zeros_nf)

    o1 = layer(x, W1)
    h1 = _make_bn_relu(n, W1.shape[1] // 128)(
        o1, b1.reshape(-1, 1, 128), g1.reshape(-1, 1, 128),
        be1.reshape(-1, 1, 128))
    o2 = layer(h1, W2)
    h2 = _make_bn_relu(n, W2.shape[1] // 128)(
        o2, b2.reshape(-1, 1, 128), g2.reshape(-1, 1, 128),
        be2.reshape(-1, 1, 128))
    o3 = layer(h2, W3)
    return _make_final(n)(o3, b3.reshape(1, 128))
